# initial kernel scaffold (unmeasured)
import jax
import jax.numpy as jnp
from jax import lax
from jax.experimental import pallas as pl
from jax.experimental.pallas import tpu as pltpu

N_DEV = 16
_GELU_C = 0.7978845608028654


def _gelu(y):
    return 0.5 * y * (1.0 + jnp.tanh(_GELU_C * (y + 0.044715 * y * y * y)))


def kernel(x, w_mat):
    m_per, k = x.shape
    _, n = w_mat.shape
    n_per = n // N_DEV
    m_out = N_DEV * m_per

    def body(x_ref, w_hbm, out_ref, w_buf, y_buf, r_buf,
             load_sems, send_sems, recv_sems):
        me = lax.axis_index("i")

        barrier = pltpu.get_barrier_semaphore()
        for d in range(N_DEV):
            @pl.when(me != d)
            def _():
                pl.semaphore_signal(
                    barrier, inc=1,
                    device_id=(d,), device_id_type=pl.DeviceIdType.MESH,
                )
        pl.semaphore_wait(barrier, N_DEV - 1)

        def w_tile_copy(g, slot):
            j = lax.rem(me + g, N_DEV)
            return pltpu.make_async_copy(
                w_hbm.at[:, pl.ds(j * n_per, n_per)],
                w_buf.at[slot],
                load_sems.at[slot],
            )

        w_tile_copy(0, 0).start()
        w_tile_copy(1, 1).start()

        send_rdmas = []
        for g in range(N_DEV):
            slot = g % 2
            j = lax.rem(me + g, N_DEV)
            w_tile_copy(g, slot).wait()
            y = jnp.dot(x_ref[...], w_buf[slot],
                        preferred_element_type=jnp.float32)
            y = _gelu(y)
            if g == 0:
                out_ref[pl.ds(me * m_per, m_per), :] = y
            else:
                y_buf[g - 1] = y.astype(jnp.bfloat16)
                rdma = pltpu.make_async_remote_copy(
                    src_ref=y_buf.at[g - 1],
                    dst_ref=r_buf.at[me],
                    send_sem=send_sems.at[g - 1],
                    recv_sem=recv_sems.at[me],
                    device_id=(j,),
                    device_id_type=pl.DeviceIdType.MESH,
                )
                rdma.start()
                send_rdmas.append(rdma)
            if g + 2 < N_DEV:
                w_tile_copy(g + 2, slot).start()

        for g in range(1, N_DEV):
            s = lax.rem(me - g + N_DEV, N_DEV)
            recv = pltpu.make_async_remote_copy(
                src_ref=y_buf.at[0],
                dst_ref=r_buf.at[s],
                send_sem=send_sems.at[0],
                recv_sem=recv_sems.at[s],
                device_id=(s,),
                device_id_type=pl.DeviceIdType.MESH,
            )
            recv.wait_recv()
            out_ref[pl.ds(s * m_per, m_per), :] = r_buf[s].astype(jnp.float32)

        for rdma in send_rdmas:
            rdma.wait_send()

    return pl.pallas_call(
        body,
        out_shape=jax.ShapeDtypeStruct((m_out, n_per), jnp.float32),
        in_specs=[
            pl.BlockSpec(memory_space=pltpu.VMEM),
            pl.BlockSpec(memory_space=pltpu.ANY),
        ],
        out_specs=pl.BlockSpec(memory_space=pltpu.VMEM),
        scratch_shapes=[
            pltpu.VMEM((2, k, n_per), w_mat.dtype),
            pltpu.VMEM((N_DEV - 1, m_per, n_per), jnp.bfloat16),
            pltpu.VMEM((N_DEV, m_per, n_per), jnp.bfloat16),
            pltpu.SemaphoreType.DMA((2,)),
            pltpu.SemaphoreType.DMA((N_DEV - 1,)),
            pltpu.SemaphoreType.DMA((N_DEV,)),
        ],
        compiler_params=pltpu.CompilerParams(collective_id=0),
    )(x, w_mat)


# baseline (device time: 103956 ns/iter reference)
import jax
import jax.numpy as jnp
from jax import lax
from jax.experimental import pallas as pl
from jax.experimental.pallas import tpu as pltpu

N_DEV = 16
_GELU_C = 0.7978845608028654


def _gelu(y):
    return 0.5 * y * (1.0 + jnp.tanh(_GELU_C * (y + 0.044715 * y * y * y)))


def kernel(x, w_mat):
    m_per, k = x.shape
    _, n = w_mat.shape
    n_per = n // N_DEV
    m_out = N_DEV * m_per

    def body(x_ref, w_hbm, out_ref, w_buf, y_buf, r_buf,
             load_sems, send_sems, recv_sems):
        me = lax.axis_index("i")

        barrier = pltpu.get_barrier_semaphore()
        for d in range(N_DEV):
            @pl.when(me != d)
            def _():
                pl.semaphore_signal(
                    barrier, inc=1,
                    device_id=(d,), device_id_type=pl.DeviceIdType.MESH,
                )
        pl.semaphore_wait(barrier, N_DEV - 1)

        def w_tile_copy(g, slot):
            j = lax.rem(me + g, N_DEV)
            return pltpu.make_async_copy(
                w_hbm.at[:, pl.ds(j * n_per, n_per)],
                w_buf.at[slot],
                load_sems.at[slot],
            )

        w_tile_copy(0, 0).start()
        w_tile_copy(1, 1).start()

        send_rdmas = []
        for g in range(N_DEV):
            slot = g % 2
            j = lax.rem(me + g, N_DEV)
            w_tile_copy(g, slot).wait()
            y = jnp.dot(x_ref[...], w_buf[slot],
                        preferred_element_type=jnp.float32)
            y = _gelu(y)
            if g == 0:
                out_ref[pl.ds(me * m_per, m_per), :] = y
            else:
                y_buf[g - 1] = y.astype(jnp.bfloat16)
                rdma = pltpu.make_async_remote_copy(
                    src_ref=y_buf.at[g - 1],
                    dst_ref=r_buf.at[me],
                    send_sem=send_sems.at[g - 1],
                    recv_sem=recv_sems.at[me],
                    device_id=(j,),
                    device_id_type=pl.DeviceIdType.MESH,
                )
                rdma.start()
                send_rdmas.append(rdma)
            if g + 2 < N_DEV:
                w_tile_copy(g + 2, slot).start()

        for g in range(1, N_DEV):
            s = lax.rem(me - g + N_DEV, N_DEV)
            recv = pltpu.make_async_remote_copy(
                src_ref=y_buf.at[0],
                dst_ref=r_buf.at[s],
                send_sem=send_sems.at[0],
                recv_sem=recv_sems.at[s],
                device_id=(s,),
                device_id_type=pl.DeviceIdType.MESH,
            )
            recv.wait_recv()
            out_ref[pl.ds(s * m_per, m_per), :] = r_buf[s].astype(jnp.float32)

        for rdma in send_rdmas:
            rdma.wait_send()

    return pl.pallas_call(
        body,
        out_shape=jax.ShapeDtypeStruct((m_out, n_per), jnp.float32),
        in_specs=[
            pl.BlockSpec(memory_space=pltpu.VMEM),
            pl.BlockSpec(memory_space=pl.ANY),
        ],
        out_specs=pl.BlockSpec(memory_space=pltpu.VMEM),
        scratch_shapes=[
            pltpu.VMEM((2, k, n_per), w_mat.dtype),
            pltpu.VMEM((N_DEV - 1, m_per, n_per), jnp.bfloat16),
            pltpu.VMEM((N_DEV, m_per, n_per), jnp.bfloat16),
            pltpu.SemaphoreType.DMA((2,)),
            pltpu.SemaphoreType.DMA((N_DEV - 1,)),
            pltpu.SemaphoreType.DMA((N_DEV,)),
        ],
        compiler_params=pltpu.CompilerParams(
            collective_id=0,
            vmem_limit_bytes=60 * 1024 * 1024,
        ),
    )(x, w_mat)


# device time: 70415 ns/iter; 1.4763x vs baseline; 1.4763x over previous
import os

import jax
import jax.numpy as jnp
from jax import lax
from jax.experimental import pallas as pl
from jax.experimental.pallas import tpu as pltpu

N_DEV = 16
_GELU_C = 0.7978845608028654
_NOSEND = os.environ.get("A2A_NOSEND", "0") == "1"
_BF16_DOT = os.environ.get("A2A_BF16_DOT", "0") == "1"


def _gelu(y):
    return 0.5 * y * (1.0 + jnp.tanh(_GELU_C * (y + 0.044715 * y * y * y)))


def kernel(x, w_mat):
    m_per, k = x.shape
    _, n = w_mat.shape
    n_per = n // N_DEV
    m_out = N_DEV * m_per

    def body(x_ref, w_hbm, out_ref, w_buf, y_buf, r_buf, x_bf, w_bf,
             load_sems, send_sems, recv_sems):
        me = lax.axis_index("i")
        if _BF16_DOT:
            x_bf[...] = x_ref[...].astype(jnp.bfloat16)

        barrier = pltpu.get_barrier_semaphore()
        for d in range(N_DEV):
            @pl.when(me != d)
            def _():
                pl.semaphore_signal(
                    barrier, inc=1,
                    device_id=(d,), device_id_type=pl.DeviceIdType.MESH,
                )
        pl.semaphore_wait(barrier, N_DEV - 1)

        def w_tile_copy(g, slot):
            j = lax.rem(me + g, N_DEV)
            return pltpu.make_async_copy(
                w_hbm.at[:, pl.ds(j * n_per, n_per)],
                w_buf.at[slot],
                load_sems.at[slot],
            )

        w_tile_copy(0, 0).start()
        w_tile_copy(1, 1).start()

        send_rdmas = []
        for g in range(N_DEV):
            slot = g % 2
            j = lax.rem(me + g, N_DEV)
            w_tile_copy(g, slot).wait()
            if _BF16_DOT:
                w_bf[...] = w_buf[slot].astype(jnp.bfloat16)
                y = jnp.dot(x_bf[...], w_bf[...],
                            preferred_element_type=jnp.float32)
            else:
                y = jnp.dot(x_ref[...], w_buf[slot],
                            preferred_element_type=jnp.float32)
            y = _gelu(y)
            if g == 0:
                out_ref[pl.ds(me * m_per, m_per), :] = y
            elif _NOSEND:
                y_buf[g - 1] = y.astype(jnp.bfloat16)
            else:
                y_buf[g - 1] = y.astype(jnp.bfloat16)
                rdma = pltpu.make_async_remote_copy(
                    src_ref=y_buf.at[g - 1],
                    dst_ref=r_buf.at[me],
                    send_sem=send_sems.at[g - 1],
                    recv_sem=recv_sems.at[me],
                    device_id=(j,),
                    device_id_type=pl.DeviceIdType.MESH,
                )
                rdma.start()
                send_rdmas.append(rdma)
            if g + 2 < N_DEV:
                w_tile_copy(g + 2, slot).start()

        for g in range(1, N_DEV) if not _NOSEND else []:
            s = lax.rem(me - g + N_DEV, N_DEV)
            recv = pltpu.make_async_remote_copy(
                src_ref=y_buf.at[0],
                dst_ref=r_buf.at[s],
                send_sem=send_sems.at[0],
                recv_sem=recv_sems.at[s],
                device_id=(s,),
                device_id_type=pl.DeviceIdType.MESH,
            )
            recv.wait_recv()
            out_ref[pl.ds(s * m_per, m_per), :] = r_buf[s].astype(jnp.float32)

        for rdma in send_rdmas:
            rdma.wait_send()

    return pl.pallas_call(
        body,
        out_shape=jax.ShapeDtypeStruct((m_out, n_per), jnp.float32),
        in_specs=[
            pl.BlockSpec(memory_space=pltpu.VMEM),
            pl.BlockSpec(memory_space=pl.ANY),
        ],
        out_specs=pl.BlockSpec(memory_space=pltpu.VMEM),
        scratch_shapes=[
            pltpu.VMEM((2, k, n_per), w_mat.dtype),
            pltpu.VMEM((N_DEV - 1, m_per, n_per), jnp.bfloat16),
            pltpu.VMEM((N_DEV, m_per, n_per), jnp.bfloat16),
            pltpu.VMEM((m_per, k), jnp.bfloat16),
            pltpu.VMEM((k, n_per), jnp.bfloat16),
            pltpu.SemaphoreType.DMA((2,)),
            pltpu.SemaphoreType.DMA((N_DEV - 1,)),
            pltpu.SemaphoreType.DMA((N_DEV,)),
        ],
        compiler_params=pltpu.CompilerParams(
            collective_id=0,
            vmem_limit_bytes=60 * 1024 * 1024,
        ),
    )(x, w_mat)
